# baseline (speedup 1.0000x reference)
import functools

import jax
import jax.numpy as jnp
from jax import lax
from jax.experimental import pallas as pl
from jax.experimental.pallas import tpu as pltpu
from jax.experimental.pallas import tpu_sc as plsc

_TOPK = 8
_M_BLK = 2048

_NEG = float("-inf")
_SENT = -2.0
_BIG = 2**31 - 1


def _topk_body(q_ref, k_ref, out_ref, qn_ref, rv_ref, ri_ref, sm_ref):
    i = pl.program_id(0)
    nt = pl.num_programs(0)
    b = q_ref.shape[0]
    mb = k_ref.shape[0]

    @pl.when(i == 0)
    def _init():
        q = q_ref[...]
        qnorm = jnp.sqrt(jnp.sum(q * q, axis=1, keepdims=True))
        qn_ref[...] = q / jnp.maximum(qnorm, 1e-12)
        rv_ref[...] = jnp.full((b, _TOPK), _NEG, jnp.float32)
        ri_ref[...] = jnp.zeros((b, _TOPK), jnp.int32)

    kk = k_ref[...]
    knorm = jnp.sqrt(jnp.sum(kk * kk, axis=1, keepdims=True))
    kn = kk / jnp.maximum(knorm, 1e-12)
    sm_ref[...] = lax.dot_general(qn_ref[...], kn, (((1,), (1,)), ((), ())),
                                  preferred_element_type=jnp.float32)
    lidx = lax.broadcasted_iota(jnp.int32, (b, mb), 1)
    li = lax.broadcasted_iota(jnp.int32, (b, _TOPK), 1)

    def cond(c):
        j, go, _rv, _ri = c
        return jnp.logical_and(j < _TOPK, go)

    def body(c):
        j, _go, rv, ri = c
        smc = sm_ref[...]
        mv = jnp.max(smc, axis=1, keepdims=True)
        im = jnp.argmax(smc, axis=1)[:, None]
        sm_ref[...] = jnp.where(lidx == im, _SENT, smc)
        gim = im + i * mb
        pos = jnp.sum((rv >= mv).astype(jnp.int32), axis=1, keepdims=True)
        rv_sh = jnp.concatenate([rv[:, :1], rv[:, :-1]], axis=1)
        ri_sh = jnp.concatenate([ri[:, :1], ri[:, :-1]], axis=1)
        nrv = jnp.where(li < pos, rv, jnp.where(li == pos, mv, rv_sh))
        nri = jnp.where(li < pos, ri, jnp.where(li == pos, gim, ri_sh))
        go = jnp.max(mv - nrv[:, 7:8]) > 0
        return (j + 1, go, nrv, nri)

    _, _, rvf, rif = lax.while_loop(
        cond, body, (0, jnp.bool_(True), rv_ref[...], ri_ref[...]))
    rv_ref[...] = rvf
    ri_ref[...] = rif

    @pl.when(i == nt - 1)
    def _flush():
        out_ref[...] = rif


def _topk_idx(query, memory_keys):
    b, d = query.shape
    cap = memory_keys.shape[0]
    mb = min(_M_BLK, cap)
    nt = cap // mb
    return pl.pallas_call(
        _topk_body,
        grid=(nt,),
        in_specs=[
            pl.BlockSpec((b, d), lambda i: (0, 0)),
            pl.BlockSpec((mb, d), lambda i: (i, 0)),
        ],
        out_specs=pl.BlockSpec((b, _TOPK), lambda i: (0, 0)),
        out_shape=jax.ShapeDtypeStruct((b, _TOPK), jnp.int32),
        scratch_shapes=[
            pltpu.VMEM((b, d), jnp.float32),
            pltpu.VMEM((b, _TOPK), jnp.float32),
            pltpu.VMEM((b, _TOPK), jnp.int32),
            pltpu.VMEM((b, mb), jnp.float32),
        ],
        compiler_params=pltpu.CompilerParams(
            dimension_semantics=("arbitrary",)),
    )(query, memory_keys)


def _sc_gather(memory_keys, memory_values, idx_flat):
    n = idx_flat.shape[0]
    d = memory_keys.shape[1]
    nw = 32
    rows_pw = n // nw
    ch = 64
    nch = rows_pw // ch
    mesh = plsc.VectorSubcoreMesh(core_axis_name="c", subcore_axis_name="s")

    @functools.partial(
        pl.kernel,
        mesh=mesh,
        out_type=[
            jax.ShapeDtypeStruct((n, d), jnp.float32),
            jax.ShapeDtypeStruct((n, d), jnp.float32),
        ],
        scratch_types=[
            pltpu.VMEM((ch,), jnp.int32),
            pltpu.VMEM((ch, d), jnp.float32),
            pltpu.VMEM((ch, d), jnp.float32),
            pltpu.SemaphoreType.DMA,
            pltpu.SemaphoreType.DMA,
        ],
    )
    def gk(keys_hbm, values_hbm, idx_hbm, outk_hbm, outv_hbm,
           idx_v, bufk, bufv, semk, semv):
        wid = lax.axis_index("s") * 2 + lax.axis_index("c")
        base = wid * rows_pw
        for c in range(nch):
            off = base + c * ch
            pltpu.sync_copy(idx_hbm.at[pl.ds(off, ch)], idx_v)
            cpk = pltpu.async_copy(keys_hbm.at[idx_v], bufk, semk)
            cpv = pltpu.async_copy(values_hbm.at[idx_v], bufv, semv)
            cpk.wait()
            cpv.wait()
            pltpu.sync_copy(bufk, outk_hbm.at[pl.ds(off, ch)])
            pltpu.sync_copy(bufv, outv_hbm.at[pl.ds(off, ch)])

    return gk(memory_keys, memory_values, idx_flat)


def kernel(query, memory_keys, memory_values, k):
    b, d = query.shape
    topk = min(8, memory_keys.shape[0])
    idx = _topk_idx(query, memory_keys)
    rk, rv = _sc_gather(memory_keys, memory_values, idx.reshape(-1))
    return rk.reshape(b, topk, d), rv.reshape(b, topk, d)


# speedup vs baseline: 1.2269x; 1.2269x over previous
import functools

import jax
import jax.numpy as jnp
from jax import lax
from jax.experimental import pallas as pl
from jax.experimental.pallas import tpu as pltpu
from jax.experimental.pallas import tpu_sc as plsc

_TOPK = 8
_M_BLK = 2048

_NEG = float("-inf")
_SENT = -2.0
_BIG = 2**31 - 1


def _topk_body(q_ref, k_ref, out_ref, qn_ref, rv_ref, ri_ref, sm_ref):
    i = pl.program_id(0)
    nt = pl.num_programs(0)
    b = q_ref.shape[0]
    mb = k_ref.shape[0]

    @pl.when(i == 0)
    def _init():
        q = q_ref[...]
        qnorm = jnp.sqrt(jnp.sum(q * q, axis=1, keepdims=True))
        qn_ref[...] = q / jnp.maximum(qnorm, 1e-12)
        rv_ref[...] = jnp.full((b, _TOPK), _NEG, jnp.float32)
        ri_ref[...] = jnp.zeros((b, _TOPK), jnp.int32)

    kk = k_ref[...]
    knorm = jnp.sqrt(jnp.sum(kk * kk, axis=1, keepdims=True))
    kn = kk / jnp.maximum(knorm, 1e-12)
    sm_ref[...] = lax.dot_general(qn_ref[...], kn, (((1,), (1,)), ((), ())),
                                  preferred_element_type=jnp.float32)
    lidx = lax.broadcasted_iota(jnp.int32, (b, mb), 1)
    li = lax.broadcasted_iota(jnp.int32, (b, _TOPK), 1)

    def cond(c):
        j, go, _rv, _ri = c
        return jnp.logical_and(j < _TOPK, go)

    def body(c):
        j, _go, rv, ri = c
        smc = sm_ref[...]
        mv = jnp.max(smc, axis=1, keepdims=True)
        im = jnp.min(jnp.where(smc == mv, lidx, _BIG), axis=1, keepdims=True)
        sm_ref[...] = jnp.where(lidx == im, _SENT, smc)
        gim = im + i * mb
        pos = jnp.sum((rv >= mv).astype(jnp.int32), axis=1, keepdims=True)
        rv_sh = jnp.concatenate([rv[:, :1], rv[:, :-1]], axis=1)
        ri_sh = jnp.concatenate([ri[:, :1], ri[:, :-1]], axis=1)
        nrv = jnp.where(li < pos, rv, jnp.where(li == pos, mv, rv_sh))
        nri = jnp.where(li < pos, ri, jnp.where(li == pos, gim, ri_sh))
        go = jnp.max(mv - nrv[:, 7:8]) > 0
        return (j + 1, go, nrv, nri)

    _, _, rvf, rif = lax.while_loop(
        cond, body, (0, jnp.bool_(True), rv_ref[...], ri_ref[...]))
    rv_ref[...] = rvf
    ri_ref[...] = rif

    @pl.when(i == nt - 1)
    def _flush():
        out_ref[...] = rif


def _topk_idx(query, memory_keys):
    b, d = query.shape
    cap = memory_keys.shape[0]
    mb = min(_M_BLK, cap)
    nt = cap // mb
    return pl.pallas_call(
        _topk_body,
        grid=(nt,),
        in_specs=[
            pl.BlockSpec((b, d), lambda i: (0, 0)),
            pl.BlockSpec((mb, d), lambda i: (i, 0)),
        ],
        out_specs=pl.BlockSpec((b, _TOPK), lambda i: (0, 0)),
        out_shape=jax.ShapeDtypeStruct((b, _TOPK), jnp.int32),
        scratch_shapes=[
            pltpu.VMEM((b, d), jnp.float32),
            pltpu.VMEM((b, _TOPK), jnp.float32),
            pltpu.VMEM((b, _TOPK), jnp.int32),
            pltpu.VMEM((b, mb), jnp.float32),
        ],
        compiler_params=pltpu.CompilerParams(
            dimension_semantics=("arbitrary",)),
    )(query, memory_keys)


def _sc_gather(memory_keys, memory_values, idx_flat):
    n = idx_flat.shape[0]
    d = memory_keys.shape[1]
    nw = 32
    rows_pw = n // nw
    ch = 64
    nch = rows_pw // ch
    mesh = plsc.VectorSubcoreMesh(core_axis_name="c", subcore_axis_name="s")

    @functools.partial(
        pl.kernel,
        mesh=mesh,
        out_type=[
            jax.ShapeDtypeStruct((n, d), jnp.float32),
            jax.ShapeDtypeStruct((n, d), jnp.float32),
        ],
        scratch_types=[
            pltpu.VMEM((ch,), jnp.int32),
            pltpu.VMEM((ch, d), jnp.float32),
            pltpu.VMEM((ch, d), jnp.float32),
            pltpu.SemaphoreType.DMA,
            pltpu.SemaphoreType.DMA,
        ],
    )
    def gk(keys_hbm, values_hbm, idx_hbm, outk_hbm, outv_hbm,
           idx_v, bufk, bufv, semk, semv):
        wid = lax.axis_index("s") * 2 + lax.axis_index("c")
        base = wid * rows_pw
        for c in range(nch):
            off = base + c * ch
            pltpu.sync_copy(idx_hbm.at[pl.ds(off, ch)], idx_v)
            cpk = pltpu.async_copy(keys_hbm.at[idx_v], bufk, semk)
            cpv = pltpu.async_copy(values_hbm.at[idx_v], bufv, semv)
            cpk.wait()
            cpv.wait()
            pltpu.sync_copy(bufk, outk_hbm.at[pl.ds(off, ch)])
            pltpu.sync_copy(bufv, outv_hbm.at[pl.ds(off, ch)])

    return gk(memory_keys, memory_values, idx_flat)


def kernel(query, memory_keys, memory_values, k):
    b, d = query.shape
    topk = min(8, memory_keys.shape[0])
    idx = _topk_idx(query, memory_keys)
    rk, rv = _sc_gather(memory_keys, memory_values, idx.reshape(-1))
    return rk.reshape(b, topk, d), rv.reshape(b, topk, d)


# speedup vs baseline: 1.2281x; 1.0010x over previous
import functools

import jax
import jax.numpy as jnp
from jax import lax
from jax.experimental import pallas as pl
from jax.experimental.pallas import tpu as pltpu
from jax.experimental.pallas import tpu_sc as plsc

_TOPK = 8
_M_BLK = 2048
_NRG = 4

_NEG = float("-inf")
_SENT = -2.0
_BIG = 2**31 - 1


def _topk_body(q_ref, k_ref, out_ref, qn_ref, rv_ref, ri_ref, sm_ref):
    i = pl.program_id(0)
    nt = pl.num_programs(0)
    b = q_ref.shape[0]
    mb = k_ref.shape[0]

    @pl.when(i == 0)
    def _init():
        q = q_ref[...]
        qnorm = jnp.sqrt(jnp.sum(q * q, axis=1, keepdims=True))
        qn_ref[...] = q / jnp.maximum(qnorm, 1e-12)
        rv_ref[...] = jnp.full((b, _TOPK), _NEG, jnp.float32)
        ri_ref[...] = jnp.zeros((b, _TOPK), jnp.int32)

    kk = k_ref[...]
    knorm = jnp.sqrt(jnp.sum(kk * kk, axis=1, keepdims=True))
    kn = kk / jnp.maximum(knorm, 1e-12)
    sm_ref[...] = lax.dot_general(qn_ref[...], kn, (((1,), (1,)), ((), ())),
                                  preferred_element_type=jnp.float32)
    rg = b // _NRG
    lidx = lax.broadcasted_iota(jnp.int32, (rg, mb), 1)
    li = lax.broadcasted_iota(jnp.int32, (rg, _TOPK), 1)

    def cond(c):
        j, go, _rv, _ri = c
        return jnp.logical_and(j < _TOPK, go)

    for g in range(_NRG):
        rs = pl.ds(g * rg, rg)

        def body(c, rs=rs):
            j, _go, rv, ri = c
            smc = sm_ref[rs, :]
            mv = jnp.max(smc, axis=1, keepdims=True)
            im = jnp.min(jnp.where(smc == mv, lidx, _BIG),
                         axis=1, keepdims=True)
            sm_ref[rs, :] = jnp.where(lidx == im, _SENT, smc)
            gim = im + i * mb
            pos = jnp.sum((rv >= mv).astype(jnp.int32), axis=1,
                          keepdims=True)
            rv_sh = jnp.concatenate([rv[:, :1], rv[:, :-1]], axis=1)
            ri_sh = jnp.concatenate([ri[:, :1], ri[:, :-1]], axis=1)
            nrv = jnp.where(li < pos, rv, jnp.where(li == pos, mv, rv_sh))
            nri = jnp.where(li < pos, ri, jnp.where(li == pos, gim, ri_sh))
            go = jnp.max(mv - nrv[:, 7:8]) > 0
            return (j + 1, go, nrv, nri)

        _, _, rvf, rif = lax.while_loop(
            cond, body, (0, jnp.bool_(True), rv_ref[rs, :], ri_ref[rs, :]))
        rv_ref[rs, :] = rvf
        ri_ref[rs, :] = rif

    @pl.when(i == nt - 1)
    def _flush():
        out_ref[...] = ri_ref[...]


def _topk_idx(query, memory_keys):
    b, d = query.shape
    cap = memory_keys.shape[0]
    mb = min(_M_BLK, cap)
    nt = cap // mb
    return pl.pallas_call(
        _topk_body,
        grid=(nt,),
        in_specs=[
            pl.BlockSpec((b, d), lambda i: (0, 0)),
            pl.BlockSpec((mb, d), lambda i: (i, 0)),
        ],
        out_specs=pl.BlockSpec((b, _TOPK), lambda i: (0, 0)),
        out_shape=jax.ShapeDtypeStruct((b, _TOPK), jnp.int32),
        scratch_shapes=[
            pltpu.VMEM((b, d), jnp.float32),
            pltpu.VMEM((b, _TOPK), jnp.float32),
            pltpu.VMEM((b, _TOPK), jnp.int32),
            pltpu.VMEM((b, mb), jnp.float32),
        ],
        compiler_params=pltpu.CompilerParams(
            dimension_semantics=("arbitrary",)),
    )(query, memory_keys)


def _sc_gather(memory_keys, memory_values, idx_flat):
    n = idx_flat.shape[0]
    d = memory_keys.shape[1]
    nw = 32
    rows_pw = n // nw
    ch = 64
    nch = rows_pw // ch
    mesh = plsc.VectorSubcoreMesh(core_axis_name="c", subcore_axis_name="s")

    @functools.partial(
        pl.kernel,
        mesh=mesh,
        out_type=[
            jax.ShapeDtypeStruct((n, d), jnp.float32),
            jax.ShapeDtypeStruct((n, d), jnp.float32),
        ],
        scratch_types=[
            pltpu.VMEM((ch,), jnp.int32),
            pltpu.VMEM((ch, d), jnp.float32),
            pltpu.VMEM((ch, d), jnp.float32),
            pltpu.SemaphoreType.DMA,
            pltpu.SemaphoreType.DMA,
        ],
    )
    def gk(keys_hbm, values_hbm, idx_hbm, outk_hbm, outv_hbm,
           idx_v, bufk, bufv, semk, semv):
        wid = lax.axis_index("s") * 2 + lax.axis_index("c")
        base = wid * rows_pw
        for c in range(nch):
            off = base + c * ch
            pltpu.sync_copy(idx_hbm.at[pl.ds(off, ch)], idx_v)
            cpk = pltpu.async_copy(keys_hbm.at[idx_v], bufk, semk)
            cpv = pltpu.async_copy(values_hbm.at[idx_v], bufv, semv)
            cpk.wait()
            cpv.wait()
            pltpu.sync_copy(bufk, outk_hbm.at[pl.ds(off, ch)])
            pltpu.sync_copy(bufv, outv_hbm.at[pl.ds(off, ch)])

    return gk(memory_keys, memory_values, idx_flat)


def kernel(query, memory_keys, memory_values, k):
    b, d = query.shape
    topk = min(8, memory_keys.shape[0])
    idx = _topk_idx(query, memory_keys)
    rk, rv = _sc_gather(memory_keys, memory_values, idx.reshape(-1))
    return rk.reshape(b, topk, d), rv.reshape(b, topk, d)
